# Initial kernel scaffold; baseline (speedup 1.0000x reference)
import jax
import jax.numpy as jnp
from jax.experimental import pallas as pl


def kernel(weights, biases, selected_anchor_points, candidate_anchor_points):
    raise NotImplementedError("write your pallas kernel here")


# speedup vs baseline: 7.4794x; 7.4794x over previous
import functools

import jax
import jax.numpy as jnp
from jax import lax
from jax.experimental import pallas as pl
from jax.experimental.pallas import tpu as pltpu
from jax.experimental.pallas import tpu_sc as plsc

NUM_FRAMES = 8
N = 1024
BATCH = 1024
N_SELECTED = 128
NBR = 32

_NC = 2
_NS = 16
_NW = _NC * _NS

@functools.cache
def _mesh():
    return plsc.VectorSubcoreMesh(
        core_axis_name="c", subcore_axis_name="s", num_cores=_NC, num_subcores=_NS
    )


@functools.cache
def _sc_build_mask():
    @functools.partial(
        pl.kernel,
        mesh=_mesh(),
        out_type=jax.ShapeDtypeStruct((NUM_FRAMES * N,), jnp.float32),
        scratch_types=[
            pltpu.VMEM((NUM_FRAMES * N,), jnp.float32),
            pltpu.VMEM((N_SELECTED,), jnp.int32),
        ],
        compiler_params=pltpu.CompilerParams(needs_layout_passes=False),
    )
    def body(ones_hbm, sel_hbm, out_hbm, table_v, idx_v):
        wid = lax.axis_index("s") * _NC + lax.axis_index("c")

        @pl.when(wid == 0)
        def _():
            pltpu.sync_copy(ones_hbm, table_v)
            pltpu.sync_copy(sel_hbm, idx_v)
            zeros = jnp.zeros((16,), jnp.float32)
            for i in range(N_SELECTED // 16):
                idx = idx_v[pl.ds(i * 16, 16)]
                plsc.store_scatter(table_v, [idx], zeros)
            pltpu.sync_copy(table_v, out_hbm)

    return body


def _tc_chain_body(w_ref, bias_ref, z_ref, out_ref, e_ref, h_ref):
    k = pl.program_id(0)

    @pl.when(k == 0)
    def _():
        e_ref[0] = z_ref[0] * bias_ref[0]

    @pl.when(k < NUM_FRAMES - 1)
    def _():
        f = k + 1
        prev = e_ref[k]
        mv = lax.dot_general(
            prev, w_ref[0], (((1,), (1,)), ((), ())),
            preferred_element_type=jnp.float32,
        )
        e_ref[f] = z_ref[f] * (bias_ref[f] + mv)

    @pl.when(k == NUM_FRAMES - 1)
    def _():
        h_ref[NUM_FRAMES - 1] = jnp.ones((1, N), jnp.float32)

    @pl.when(k >= NUM_FRAMES - 1)
    def _():
        j = 2 * (NUM_FRAMES - 1) - 1 - k
        x = z_ref[j + 1] * h_ref[j + 1]
        mv = lax.dot_general(
            x, w_ref[0], (((1,), (0,)), ((), ())),
            preferred_element_type=jnp.float32,
        )
        h_ref[j] = 1.0 + mv

    @pl.when(k == 2 * (NUM_FRAMES - 1) - 1)
    def _():
        s_base = jnp.sum(e_ref[...])
        out_ref[...] = s_base - e_ref[...] * h_ref[...]


def _tc_chain(weights, biases3, zmask3):
    n_steps = 2 * (NUM_FRAMES - 1)
    return pl.pallas_call(
        _tc_chain_body,
        grid=(n_steps,),
        in_specs=[
            pl.BlockSpec(
                (1, N, N),
                lambda k: (jnp.minimum(k, n_steps - 1 - k), 0, 0),
            ),
            pl.BlockSpec((NUM_FRAMES, 1, N), lambda k: (0, 0, 0)),
            pl.BlockSpec((NUM_FRAMES, 1, N), lambda k: (0, 0, 0)),
        ],
        out_specs=pl.BlockSpec((NUM_FRAMES, 1, N), lambda k: (0, 0, 0)),
        out_shape=jax.ShapeDtypeStruct((NUM_FRAMES, 1, N), jnp.float32),
        scratch_shapes=[
            pltpu.VMEM((NUM_FRAMES, 1, N), jnp.float32),
            pltpu.VMEM((NUM_FRAMES, 1, N), jnp.float32),
        ],
        compiler_params=pltpu.CompilerParams(
            dimension_semantics=("arbitrary",),
        ),
    )(weights, biases3, zmask3)


_B_PER_W = BATCH // _NW


@functools.cache
def _sc_gather():
    @functools.partial(
        pl.kernel,
        mesh=_mesh(),
        out_type=jax.ShapeDtypeStruct((BATCH,), jnp.float32),
        scratch_types=[
            pltpu.VMEM((_B_PER_W,), jnp.int32),
            pltpu.VMEM((_B_PER_W,), jnp.float32),
            pltpu.SemaphoreType.DMA,
        ],
    )
    def body(table_hbm, idx_hbm, out_hbm, idx_v, vals_v, sem):
        wid = lax.axis_index("s") * _NC + lax.axis_index("c")
        base = wid * _B_PER_W
        pltpu.sync_copy(idx_hbm.at[pl.ds(base, _B_PER_W)], idx_v)
        pltpu.async_copy(table_hbm.at[idx_v], vals_v, sem).wait()
        pltpu.sync_copy(vals_v, out_hbm.at[pl.ds(base, _B_PER_W)])

    return body


def kernel(weights, biases, selected_anchor_points, candidate_anchor_points):
    sel = selected_anchor_points.astype(jnp.int32)
    cand = candidate_anchor_points.astype(jnp.int32)
    sel_flat = sel[:, 0] * N + sel[:, 1] * NBR + sel[:, 2]
    cand_flat = cand[:, 0] * N + cand[:, 1] * NBR + cand[:, 2]

    ones = jnp.ones((NUM_FRAMES * N,), jnp.float32)
    zmask = _sc_build_mask()(ones, sel_flat)
    table = _tc_chain(
        weights,
        biases.reshape(NUM_FRAMES, 1, N),
        zmask.reshape(NUM_FRAMES, 1, N),
    )
    return _sc_gather()(table.reshape(NUM_FRAMES * N), cand_flat)


# speedup vs baseline: 9.8812x; 1.3211x over previous
import functools

import jax
import jax.numpy as jnp
from jax import lax
from jax.experimental import pallas as pl
from jax.experimental.pallas import tpu as pltpu
from jax.experimental.pallas import tpu_sc as plsc

NUM_FRAMES = 8
N = 1024
BATCH = 1024
N_SELECTED = 128
NBR = 32

_NC = 2
_NS = 16
_NW = _NC * _NS


@functools.cache
def _mesh():
    return plsc.VectorSubcoreMesh(
        core_axis_name="c", subcore_axis_name="s", num_cores=_NC, num_subcores=_NS
    )


def _tc_chain_body(sel_ref, bias_ref, w_hbm, out_ref, wv, sems):
    for f in range(NUM_FRAMES - 1):
        pltpu.make_async_copy(w_hbm.at[f], wv.at[f], sems.at[f]).start()

    fr = lax.broadcasted_iota(jnp.int32, (NUM_FRAMES, N), 0)
    ln = lax.broadcasted_iota(jnp.int32, (NUM_FRAMES, N), 1)
    flat = fr * N + ln

    def mk(i, z):
        s = sel_ref[0, i]
        return jnp.where(flat == s, 0.0, z)

    z = lax.fori_loop(0, N_SELECTED, mk, jnp.ones((NUM_FRAMES, N), jnp.float32))

    e = [None] * NUM_FRAMES
    e[0] = z[0:1] * bias_ref[0:1]
    for f in range(1, NUM_FRAMES):
        pltpu.make_async_copy(w_hbm.at[f - 1], wv.at[f - 1], sems.at[f - 1]).wait()
        mv = lax.dot_general(
            e[f - 1], wv[f - 1], (((1,), (1,)), ((), ())),
            preferred_element_type=jnp.float32,
        )
        e[f] = z[f:f + 1] * (bias_ref[f:f + 1] + mv)

    h = [None] * NUM_FRAMES
    h[NUM_FRAMES - 1] = jnp.ones((1, N), jnp.float32)
    for f in range(NUM_FRAMES - 2, -1, -1):
        x = z[f + 1:f + 2] * h[f + 1]
        mv = lax.dot_general(
            x, wv[f], (((1,), (0,)), ((), ())),
            preferred_element_type=jnp.float32,
        )
        h[f] = 1.0 + mv

    ee = jnp.concatenate(e, axis=0)
    hh = jnp.concatenate(h, axis=0)
    out_ref[...] = jnp.sum(ee) - ee * hh


def _tc_chain(sel_flat2, biases, weights):
    return pl.pallas_call(
        _tc_chain_body,
        in_specs=[
            pl.BlockSpec(memory_space=pltpu.SMEM),
            pl.BlockSpec(memory_space=pltpu.VMEM),
            pl.BlockSpec(memory_space=pl.ANY),
        ],
        out_specs=pl.BlockSpec(memory_space=pltpu.VMEM),
        out_shape=jax.ShapeDtypeStruct((NUM_FRAMES, N), jnp.float32),
        scratch_shapes=[
            pltpu.VMEM((NUM_FRAMES - 1, N, N), jnp.float32),
            pltpu.SemaphoreType.DMA((NUM_FRAMES - 1,)),
        ],
    )(sel_flat2, biases, weights)


_B_PER_W = BATCH // _NW


@functools.cache
def _sc_gather():
    @functools.partial(
        pl.kernel,
        mesh=_mesh(),
        out_type=jax.ShapeDtypeStruct((BATCH,), jnp.float32),
        scratch_types=[
            pltpu.VMEM((_B_PER_W,), jnp.int32),
            pltpu.VMEM((_B_PER_W,), jnp.float32),
            pltpu.SemaphoreType.DMA,
        ],
        compiler_params=pltpu.CompilerParams(needs_layout_passes=False),
    )
    def body(table_hbm, idx_hbm, out_hbm, idx_v, vals_v, sem):
        wid = lax.axis_index("s") * _NC + lax.axis_index("c")
        base = wid * _B_PER_W
        pltpu.sync_copy(idx_hbm.at[pl.ds(base, _B_PER_W)], idx_v)
        pltpu.async_copy(table_hbm.at[idx_v], vals_v, sem).wait()
        pltpu.sync_copy(vals_v, out_hbm.at[pl.ds(base, _B_PER_W)])

    return body


def kernel(weights, biases, selected_anchor_points, candidate_anchor_points):
    sel = selected_anchor_points.astype(jnp.int32)
    cand = candidate_anchor_points.astype(jnp.int32)
    sel_flat = (sel[:, 0] * N + sel[:, 1] * NBR + sel[:, 2]).reshape(1, N_SELECTED)
    cand_flat = cand[:, 0] * N + cand[:, 1] * NBR + cand[:, 2]

    table = _tc_chain(sel_flat, biases, weights)
    return _sc_gather()(table.reshape(NUM_FRAMES * N), cand_flat)
